# Initial kernel scaffold; baseline (speedup 1.0000x reference)
import jax
import jax.numpy as jnp
from jax.experimental import pallas as pl


def kernel(map_tensor, piece_tensor, W_in, b_in, Wg1, bg1, Wg2, bg2, W_out, b_out, P1, p1b, P2, p2b, P3, p3b, V1, v1b, V2, v2b, V3, v3b):
    raise NotImplementedError("write your pallas kernel here")


# speedup vs baseline: 18.1849x; 18.1849x over previous
import jax
import jax.numpy as jnp
from jax import lax
from jax.experimental import pallas as pl

_S = 12
_CELLS = _S * _S
_BB = 32
_NEG = jnp.finfo(jnp.float32).min


def _elu(v):
    return jnp.where(v > 0, v, jnp.exp(jnp.minimum(v, 0.0)) - 1.0)


def _fused_kernel(map_ref, piece_ref, gum_ref,
                  W_in_ref, b_in_ref, Wg1_ref, bg1_ref, Wg2_ref, bg2_ref,
                  W_out_ref, b_out_ref, P1_ref, p1b_ref, P2_ref, p2b_ref,
                  P3_ref, p3b_ref, V1_ref, v1b_ref, V2_ref, v2b_ref,
                  V3_ref, v3b_ref,
                  logits_ref, act_ref, val_ref):
    BB = _BB
    N = BB * _CELLS
    H = 128

    r = lax.broadcasted_iota(jnp.int32, (N, 1), 0)
    c = r % _CELLS
    y = c % _S
    x = c // _S
    f32 = jnp.float32
    my1 = (y < _S - 1).astype(f32)
    my0 = (y > 0).astype(f32)
    mx1 = (x < _S - 1).astype(f32)
    mx0 = (x > 0).astype(f32)
    deg = 1.0 + my1 + my0 + mx1 + mx0

    z1 = jnp.zeros((1, H), f32)
    z12 = jnp.zeros((_S, H), f32)

    def nbr_sum(h):
        up = jnp.concatenate([h[1:], z1], axis=0) * my1
        dn = jnp.concatenate([z1, h[:-1]], axis=0) * my0
        xp = jnp.concatenate([h[_S:], z12], axis=0) * mx1
        xm = jnp.concatenate([z12, h[:-_S]], axis=0) * mx0
        return up + dn + xp + xm

    W_in = W_in_ref[...]
    b_in = b_in_ref[...]
    Wg1 = Wg1_ref[...]
    bg1 = bg1_ref[...]

    h1 = _elu(jnp.dot(map_ref[...], W_in, preferred_element_type=f32) + b_in)
    h1_meta = _elu(b_in)

    sum1 = h1.reshape(BB, _CELLS, H).sum(axis=1)
    agg1 = (nbr_sum(h1) + h1_meta) / deg
    agg1_meta = sum1 * (1.0 / _CELLS)
    h2 = _elu(jnp.dot(h1 + agg1, Wg1, preferred_element_type=f32) + bg1)
    h2_meta = _elu(jnp.dot(h1_meta + agg1_meta, Wg1,
                           preferred_element_type=f32) + bg1)

    sum2 = h2.reshape(BB, _CELLS, H).sum(axis=1)
    meta_r2 = h2_meta + sum2 * (1.0 / _CELLS)
    mb = jnp.broadcast_to(h2_meta[:, None, :], (BB, _CELLS, H)).reshape(N, H)
    cells_r2 = h2 + (nbr_sum(h2) + mb) / deg

    pt = piece_ref[...]
    jf = jnp.floor(pt[:, 1:2]) * _S + jnp.floor(pt[:, 2:3])
    jb = jnp.broadcast_to(jf[:, None, :], (BB, _CELLS, 1)).reshape(N, 1)
    sel = (c.astype(f32) == jb).astype(f32)
    rj = (cells_r2 * sel).reshape(BB, _CELLS, H).sum(axis=1)

    st = jnp.concatenate([rj, meta_r2], axis=0)
    h3 = _elu(jnp.dot(st, Wg2_ref[...], preferred_element_type=f32)
              + bg2_ref[...])
    emb = jnp.dot(h3, W_out_ref[...], preferred_element_type=f32) \
        + b_out_ref[...]
    cs = emb[:BB]
    ms = emb[BB:]

    p0 = jnp.floor(pt[:, 0:1])
    oh3 = jnp.concatenate([(p0 == 0.0).astype(f32), (p0 == 1.0).astype(f32),
                           (p0 == 2.0).astype(f32)], axis=1)
    P1 = P1_ref[...]
    hp = _elu(jnp.dot(cs, P1[:128], preferred_element_type=f32)
              + jnp.dot(oh3, P1[128:131], preferred_element_type=f32)
              + p1b_ref[...])
    hp = _elu(jnp.dot(hp, P2_ref[...], preferred_element_type=f32)
              + p2b_ref[...])
    logits = jnp.dot(hp, P3_ref[...], preferred_element_type=f32) \
        + p3b_ref[...]
    lm = jnp.where(pt[:, 3:16] != 0.0, logits, _NEG)
    logits_ref[...] = lm

    am = lm + gum_ref[...]
    mx = jnp.max(am, axis=1, keepdims=True)
    li = lax.broadcasted_iota(jnp.int32, am.shape, 1)
    act_ref[...] = jnp.min(jnp.where(am == mx, li, am.shape[1]),
                           axis=1, keepdims=True).astype(jnp.int32)

    hv = _elu(jnp.dot(ms, V1_ref[...], preferred_element_type=f32)
              + v1b_ref[...])
    hv = _elu(jnp.dot(hv, V2_ref[...], preferred_element_type=f32)
              + v2b_ref[...])
    val_ref[...] = jnp.tanh(
        jnp.dot(hv, V3_ref[...], preferred_element_type=f32) + v3b_ref[...])


def kernel(map_tensor, piece_tensor, W_in, b_in, Wg1, bg1, Wg2, bg2, W_out,
           b_out, P1, p1b, P2, p2b, P3, p3b, V1, v1b, V2, v2b, V3, v3b):
    B = map_tensor.shape[0]
    assert B % _BB == 0
    n_act = P3.shape[1]
    map_flat = map_tensor.reshape(B * _CELLS, map_tensor.shape[-1])
    gum = jax.random.gumbel(jax.random.key(1), (B, n_act), jnp.float32)

    row2 = lambda v: v.reshape(1, -1)
    weights = (W_in, row2(b_in), Wg1, row2(bg1), Wg2, row2(bg2),
               W_out, row2(b_out), P1, row2(p1b), P2, row2(p2b),
               P3, row2(p3b), V1, row2(v1b), V2, row2(v2b),
               V3, row2(v3b))

    w_specs = [pl.BlockSpec(w.shape, lambda i: (0, 0)) for w in weights]
    grid = (B // _BB,)
    logits_m, act2d, value = pl.pallas_call(
        _fused_kernel,
        grid=grid,
        in_specs=[
            pl.BlockSpec((_BB * _CELLS, map_flat.shape[1]), lambda i: (i, 0)),
            pl.BlockSpec((_BB, piece_tensor.shape[1]), lambda i: (i, 0)),
            pl.BlockSpec((_BB, n_act), lambda i: (i, 0)),
        ] + w_specs,
        out_specs=[
            pl.BlockSpec((_BB, n_act), lambda i: (i, 0)),
            pl.BlockSpec((_BB, 1), lambda i: (i, 0)),
            pl.BlockSpec((_BB, 1), lambda i: (i, 0)),
        ],
        out_shape=[
            jax.ShapeDtypeStruct((B, n_act), jnp.float32),
            jax.ShapeDtypeStruct((B, 1), jnp.int32),
            jax.ShapeDtypeStruct((B, 1), jnp.float32),
        ],
    )(map_flat, piece_tensor, gum, *weights)
    return (act2d.reshape(B), logits_m, value)


# speedup vs baseline: 20.8673x; 1.1475x over previous
import jax
import jax.numpy as jnp
from jax import lax
from jax.experimental import pallas as pl
from jax.experimental.pallas import tpu as pltpu

_S = 12
_CELLS = _S * _S
_BB = 32
_NEG = jnp.finfo(jnp.float32).min


def _elu(v):
    return jnp.maximum(v, 0.0) + jnp.exp(jnp.minimum(v, 0.0)) - 1.0


def _fused_kernel(map_ref, piece_ref, gum_ref, j_ref,
                  W_in_ref, b_in_ref, Wg1_ref, bg1_ref, Wg2_ref, bg2_ref,
                  W_out_ref, b_out_ref, P1_ref, p1b_ref, P2_ref, p2b_ref,
                  P3_ref, p3b_ref, V1_ref, v1b_ref, V2_ref, v2b_ref,
                  V3_ref, v3b_ref,
                  logits_ref, act_ref, val_ref,
                  h2_ref):
    BB = _BB
    N = BB * _CELLS
    H = 128

    r = lax.broadcasted_iota(jnp.int32, (N, 1), 0)
    c = r % _CELLS
    y = c % _S
    x = c // _S
    f32 = jnp.float32
    my1 = (y < _S - 1).astype(f32)
    my0 = (y > 0).astype(f32)
    mx1 = (x < _S - 1).astype(f32)
    mx0 = (x > 0).astype(f32)
    rdeg = 1.0 / (1.0 + my1 + my0 + mx1 + mx0)

    z1 = jnp.zeros((1, H), f32)
    z12 = jnp.zeros((_S, H), f32)

    W_in = W_in_ref[...]
    b_in = b_in_ref[...]
    Wg1 = Wg1_ref[...]
    bg1 = bg1_ref[...]

    h1 = _elu(jnp.dot(map_ref[...], W_in, preferred_element_type=f32) + b_in)
    h1_meta = _elu(b_in)

    up = jnp.concatenate([h1[1:], z1], axis=0) * my1
    dn = jnp.concatenate([z1, h1[:-1]], axis=0) * my0
    xp = jnp.concatenate([h1[_S:], z12], axis=0) * mx1
    xm = jnp.concatenate([z12, h1[:-_S]], axis=0) * mx0
    rows1 = h1 + (up + dn + xp + xm + h1_meta) * rdeg
    sum1 = h1.reshape(BB, _CELLS, H).sum(axis=1)
    h2 = _elu(jnp.dot(rows1, Wg1, preferred_element_type=f32) + bg1)
    h2_meta = _elu(jnp.dot(h1_meta + sum1 * (1.0 / _CELLS), Wg1,
                           preferred_element_type=f32) + bg1)
    h2_ref[...] = h2

    sum2 = h2.reshape(BB, _CELLS, H).sum(axis=1)
    meta_r2 = h2_meta + sum2 * (1.0 / _CELLS)

    rj_rows = []
    for g in range(BB):
        base = g * _CELLS
        jg = j_ref[g, 0]
        yg = lax.rem(jg, _S)
        xg = lax.div(jg, _S)
        ctr = h2_ref[pl.ds(base + jg, 1), :]
        gu = h2_ref[pl.ds(base + jnp.minimum(jg + 1, _CELLS - 1), 1), :]
        gd = h2_ref[pl.ds(base + jnp.maximum(jg - 1, 0), 1), :]
        gr = h2_ref[pl.ds(base + jnp.minimum(jg + _S, _CELLS - 1), 1), :]
        gl = h2_ref[pl.ds(base + jnp.maximum(jg - _S, 0), 1), :]
        fu = (yg < _S - 1).astype(f32)
        fd = (yg > 0).astype(f32)
        fr = (xg < _S - 1).astype(f32)
        fl = (xg > 0).astype(f32)
        nb = gu * fu + gd * fd + gr * fr + gl * fl + h2_meta[g:g + 1, :]
        rj_rows.append(ctr + nb * (1.0 / (1.0 + fu + fd + fr + fl)))
    rj = jnp.concatenate(rj_rows, axis=0)

    st = jnp.concatenate([rj, meta_r2], axis=0)
    h3 = _elu(jnp.dot(st, Wg2_ref[...], preferred_element_type=f32)
              + bg2_ref[...])
    emb = jnp.dot(h3, W_out_ref[...], preferred_element_type=f32) \
        + b_out_ref[...]
    cs = emb[:BB]
    ms = emb[BB:]

    pt = piece_ref[...]
    p0 = jnp.floor(pt[:, 0:1])
    oh3 = jnp.concatenate([(p0 == 0.0).astype(f32), (p0 == 1.0).astype(f32),
                           (p0 == 2.0).astype(f32)], axis=1)
    P1 = P1_ref[...]
    hp = _elu(jnp.dot(cs, P1[:128], preferred_element_type=f32)
              + jnp.dot(oh3, P1[128:131], preferred_element_type=f32)
              + p1b_ref[...])
    hp = _elu(jnp.dot(hp, P2_ref[...], preferred_element_type=f32)
              + p2b_ref[...])
    logits = jnp.dot(hp, P3_ref[...], preferred_element_type=f32) \
        + p3b_ref[...]
    lm = jnp.where(pt[:, 3:16] != 0.0, logits, _NEG)
    logits_ref[...] = lm

    am = lm + gum_ref[...]
    mx = jnp.max(am, axis=1, keepdims=True)
    li = lax.broadcasted_iota(jnp.int32, am.shape, 1)
    act_ref[...] = jnp.min(jnp.where(am == mx, li, am.shape[1]),
                           axis=1, keepdims=True).astype(jnp.int32)

    hv = _elu(jnp.dot(ms, V1_ref[...], preferred_element_type=f32)
              + v1b_ref[...])
    hv = _elu(jnp.dot(hv, V2_ref[...], preferred_element_type=f32)
              + v2b_ref[...])
    val_ref[...] = jnp.tanh(
        jnp.dot(hv, V3_ref[...], preferred_element_type=f32) + v3b_ref[...])


def kernel(map_tensor, piece_tensor, W_in, b_in, Wg1, bg1, Wg2, bg2, W_out,
           b_out, P1, p1b, P2, p2b, P3, p3b, V1, v1b, V2, v2b, V3, v3b):
    B = map_tensor.shape[0]
    assert B % _BB == 0
    n_act = P3.shape[1]
    map_flat = map_tensor.reshape(B * _CELLS, map_tensor.shape[-1])
    gum = jax.random.gumbel(jax.random.key(1), (B, n_act), jnp.float32)
    pos = piece_tensor[:, 1:3].astype(jnp.int32)
    jarr = (pos[:, 0] * _S + pos[:, 1]).reshape(B, 1)

    row2 = lambda v: v.reshape(1, -1)
    weights = (W_in, row2(b_in), Wg1, row2(bg1), Wg2, row2(bg2),
               W_out, row2(b_out), P1, row2(p1b), P2, row2(p2b),
               P3, row2(p3b), V1, row2(v1b), V2, row2(v2b),
               V3, row2(v3b))

    w_specs = [pl.BlockSpec(w.shape, lambda i: (0, 0)) for w in weights]
    grid = (B // _BB,)
    logits_m, act2d, value = pl.pallas_call(
        _fused_kernel,
        grid=grid,
        in_specs=[
            pl.BlockSpec((_BB * _CELLS, map_flat.shape[1]), lambda i: (i, 0)),
            pl.BlockSpec((_BB, piece_tensor.shape[1]), lambda i: (i, 0)),
            pl.BlockSpec((_BB, n_act), lambda i: (i, 0)),
            pl.BlockSpec((_BB, 1), lambda i: (i, 0),
                         memory_space=pltpu.SMEM),
        ] + w_specs,
        out_specs=[
            pl.BlockSpec((_BB, n_act), lambda i: (i, 0)),
            pl.BlockSpec((_BB, 1), lambda i: (i, 0)),
            pl.BlockSpec((_BB, 1), lambda i: (i, 0)),
        ],
        out_shape=[
            jax.ShapeDtypeStruct((B, n_act), jnp.float32),
            jax.ShapeDtypeStruct((B, 1), jnp.int32),
            jax.ShapeDtypeStruct((B, 1), jnp.float32),
        ],
        scratch_shapes=[pltpu.VMEM((_BB * _CELLS, 128), jnp.float32)],
    )(map_flat, piece_tensor, gum, jarr, *weights)
    return (act2d.reshape(B), logits_m, value)


# speedup vs baseline: 29.8844x; 1.4321x over previous
import jax
import jax.numpy as jnp
import numpy as np
from jax import lax
from jax.experimental import pallas as pl
from jax.experimental.pallas import tpu as pltpu

_S = 12
_CELLS = _S * _S
_BB = 32
_NEG = jnp.finfo(jnp.float32).min


def _stencil_consts():
    c = np.arange(_CELLS)
    y, x = c % _S, c // _S
    my1 = (y < _S - 1).astype(np.float32)
    my0 = (y > 0).astype(np.float32)
    mx1 = (x < _S - 1).astype(np.float32)
    mx0 = (x > 0).astype(np.float32)
    rdeg = 1.0 / (1.0 + my1 + my0 + mx1 + mx0)
    rows = np.concatenate([my1 * rdeg, my0 * rdeg, mx1 * rdeg, mx0 * rdeg,
                           rdeg]).astype(np.float32)
    return np.repeat(rows[:, None], 128, axis=1)


_STC = _stencil_consts()


def _elu(v):
    return jnp.maximum(v, 0.0) + jnp.exp(jnp.minimum(v, 0.0)) - 1.0


def _fused_kernel(map_ref, piece_ref, gum_ref, j_ref, stc_ref,
                  W_in_ref, b_in_ref, Wg1_ref, bg1_ref, Wg2_ref, bg2_ref,
                  W_out_ref, b_out_ref, P1_ref, p1b_ref, P2_ref, p2b_ref,
                  P3_ref, p3b_ref, V1_ref, v1b_ref, V2_ref, v2b_ref,
                  V3_ref, v3b_ref,
                  logits_ref, act_ref, val_ref,
                  h2_ref):
    BB = _BB
    N = BB * _CELLS
    H = 128
    f32 = jnp.float32

    au = stc_ref[0:_CELLS].reshape(1, _CELLS, H)
    ad = stc_ref[_CELLS:2 * _CELLS].reshape(1, _CELLS, H)
    ap = stc_ref[2 * _CELLS:3 * _CELLS].reshape(1, _CELLS, H)
    am_ = stc_ref[3 * _CELLS:4 * _CELLS].reshape(1, _CELLS, H)
    rdt = stc_ref[4 * _CELLS:5 * _CELLS].reshape(1, _CELLS, H)

    z1 = jnp.zeros((1, H), f32)
    z12 = jnp.zeros((_S, H), f32)

    W_in = W_in_ref[...]
    b_in = b_in_ref[...]
    Wg1 = Wg1_ref[...]
    bg1 = bg1_ref[...]

    h1 = _elu(jnp.dot(map_ref[...], W_in, preferred_element_type=f32) + b_in)
    h1_meta = _elu(b_in)

    up = jnp.concatenate([h1[1:], z1], axis=0).reshape(BB, _CELLS, H)
    dn = jnp.concatenate([z1, h1[:-1]], axis=0).reshape(BB, _CELLS, H)
    xp = jnp.concatenate([h1[_S:], z12], axis=0).reshape(BB, _CELLS, H)
    xm = jnp.concatenate([z12, h1[:-_S]], axis=0).reshape(BB, _CELLS, H)
    m_term = h1_meta.reshape(1, 1, H) * rdt
    rows1 = (h1.reshape(BB, _CELLS, H)
             + up * au + dn * ad + xp * ap + xm * am_ + m_term).reshape(N, H)
    sum1 = h1.reshape(BB, _CELLS, H).sum(axis=1)
    h2 = _elu(jnp.dot(rows1, Wg1, preferred_element_type=f32) + bg1)
    h2_meta = _elu(jnp.dot(h1_meta + sum1 * (1.0 / _CELLS), Wg1,
                           preferred_element_type=f32) + bg1)
    h2_ref[...] = h2

    sum2 = h2.reshape(BB, _CELLS, H).sum(axis=1)
    meta_r2 = h2_meta + sum2 * (1.0 / _CELLS)

    rj_rows = []
    for g in range(BB):
        base = g * _CELLS
        jg = j_ref[g, 0]
        yg = lax.rem(jg, _S)
        xg = lax.div(jg, _S)
        ctr = h2_ref[pl.ds(base + jg, 1), :]
        gu = h2_ref[pl.ds(base + jnp.minimum(jg + 1, _CELLS - 1), 1), :]
        gd = h2_ref[pl.ds(base + jnp.maximum(jg - 1, 0), 1), :]
        gr = h2_ref[pl.ds(base + jnp.minimum(jg + _S, _CELLS - 1), 1), :]
        gl = h2_ref[pl.ds(base + jnp.maximum(jg - _S, 0), 1), :]
        fu = (yg < _S - 1).astype(f32)
        fd = (yg > 0).astype(f32)
        fr = (xg < _S - 1).astype(f32)
        fl = (xg > 0).astype(f32)
        nb = gu * fu + gd * fd + gr * fr + gl * fl + h2_meta[g:g + 1, :]
        rj_rows.append(ctr + nb * (1.0 / (1.0 + fu + fd + fr + fl)))
    rj = jnp.concatenate(rj_rows, axis=0)

    st = jnp.concatenate([rj, meta_r2], axis=0)
    h3 = _elu(jnp.dot(st, Wg2_ref[...], preferred_element_type=f32)
              + bg2_ref[...])
    emb = jnp.dot(h3, W_out_ref[...], preferred_element_type=f32) \
        + b_out_ref[...]
    cs = emb[:BB]
    ms = emb[BB:]

    pt = piece_ref[...]
    p0 = jnp.floor(pt[:, 0:1])
    oh3 = jnp.concatenate([(p0 == 0.0).astype(f32), (p0 == 1.0).astype(f32),
                           (p0 == 2.0).astype(f32)], axis=1)
    P1 = P1_ref[...]
    hp = _elu(jnp.dot(cs, P1[:128], preferred_element_type=f32)
              + jnp.dot(oh3, P1[128:131], preferred_element_type=f32)
              + p1b_ref[...])
    hp = _elu(jnp.dot(hp, P2_ref[...], preferred_element_type=f32)
              + p2b_ref[...])
    logits = jnp.dot(hp, P3_ref[...], preferred_element_type=f32) \
        + p3b_ref[...]
    lm = jnp.where(pt[:, 3:16] != 0.0, logits, _NEG)
    logits_ref[...] = lm

    am = lm + gum_ref[...]
    mx = jnp.max(am, axis=1, keepdims=True)
    li = lax.broadcasted_iota(jnp.int32, am.shape, 1)
    act_ref[...] = jnp.min(jnp.where(am == mx, li, am.shape[1]),
                           axis=1, keepdims=True).astype(jnp.int32)

    hv = _elu(jnp.dot(ms, V1_ref[...], preferred_element_type=f32)
              + v1b_ref[...])
    hv = _elu(jnp.dot(hv, V2_ref[...], preferred_element_type=f32)
              + v2b_ref[...])
    val_ref[...] = jnp.tanh(
        jnp.dot(hv, V3_ref[...], preferred_element_type=f32) + v3b_ref[...])


def kernel(map_tensor, piece_tensor, W_in, b_in, Wg1, bg1, Wg2, bg2, W_out,
           b_out, P1, p1b, P2, p2b, P3, p3b, V1, v1b, V2, v2b, V3, v3b):
    B = map_tensor.shape[0]
    assert B % _BB == 0
    n_act = P3.shape[1]
    map_flat = map_tensor.reshape(B * _CELLS, map_tensor.shape[-1])
    gum = jax.random.gumbel(jax.random.key(1), (B, n_act), jnp.float32)
    pos = piece_tensor[:, 1:3].astype(jnp.int32)
    jarr = (pos[:, 0] * _S + pos[:, 1]).reshape(B, 1)

    row2 = lambda v: v.reshape(1, -1)
    weights = (W_in, row2(b_in), Wg1, row2(bg1), Wg2, row2(bg2),
               W_out, row2(b_out), P1, row2(p1b), P2, row2(p2b),
               P3, row2(p3b), V1, row2(v1b), V2, row2(v2b),
               V3, row2(v3b))

    w_specs = [pl.BlockSpec(w.shape, lambda i: (0, 0)) for w in weights]
    grid = (B // _BB,)
    logits_m, act2d, value = pl.pallas_call(
        _fused_kernel,
        grid=grid,
        in_specs=[
            pl.BlockSpec((_BB * _CELLS, map_flat.shape[1]), lambda i: (i, 0)),
            pl.BlockSpec((_BB, piece_tensor.shape[1]), lambda i: (i, 0)),
            pl.BlockSpec((_BB, n_act), lambda i: (i, 0)),
            pl.BlockSpec((_BB, 1), lambda i: (i, 0),
                         memory_space=pltpu.SMEM),
            pl.BlockSpec(_STC.shape, lambda i: (0, 0)),
        ] + w_specs,
        out_specs=[
            pl.BlockSpec((_BB, n_act), lambda i: (i, 0)),
            pl.BlockSpec((_BB, 1), lambda i: (i, 0)),
            pl.BlockSpec((_BB, 1), lambda i: (i, 0)),
        ],
        out_shape=[
            jax.ShapeDtypeStruct((B, n_act), jnp.float32),
            jax.ShapeDtypeStruct((B, 1), jnp.int32),
            jax.ShapeDtypeStruct((B, 1), jnp.float32),
        ],
        scratch_shapes=[pltpu.VMEM((_BB * _CELLS, 128), jnp.float32)],
    )(map_flat, piece_tensor, gum, jarr, jnp.asarray(_STC), *weights)
    return (act2d.reshape(B), logits_m, value)


# speedup vs baseline: 31.7148x; 1.0612x over previous
import jax
import jax.numpy as jnp
import numpy as np
from jax import lax
from jax.experimental import pallas as pl
from jax.experimental.pallas import tpu as pltpu

_S = 12
_CELLS = _S * _S
_BB = 64
_NEG = jnp.finfo(jnp.float32).min


def _stencil_consts():
    c = np.arange(_CELLS)
    y, x = c % _S, c // _S
    my1 = (y < _S - 1).astype(np.float32)
    my0 = (y > 0).astype(np.float32)
    mx1 = (x < _S - 1).astype(np.float32)
    mx0 = (x > 0).astype(np.float32)
    rdeg = 1.0 / (1.0 + my1 + my0 + mx1 + mx0)
    rows = np.concatenate([my1 * rdeg, my0 * rdeg, mx1 * rdeg, mx0 * rdeg,
                           rdeg]).astype(np.float32)
    return np.repeat(rows[:, None], 128, axis=1)


_STC = _stencil_consts()


def _elu(v):
    return jnp.maximum(v, 0.0) + jnp.exp(jnp.minimum(v, 0.0)) - 1.0


def _fused_kernel(map_ref, piece_ref, gum_ref, j_ref, stc_ref,
                  W_in_ref, b_in_ref, Wg1_ref, bg1_ref, Wg2_ref, bg2_ref,
                  W_out_ref, b_out_ref, P1_ref, p1b_ref, P2_ref, p2b_ref,
                  P3_ref, p3b_ref, V1_ref, v1b_ref, V2_ref, v2b_ref,
                  V3_ref, v3b_ref,
                  logits_ref, act_ref, val_ref,
                  h2_ref):
    BB = _BB
    N = BB * _CELLS
    H = 128
    f32 = jnp.float32

    au = stc_ref[0:_CELLS].reshape(1, _CELLS, H)
    ad = stc_ref[_CELLS:2 * _CELLS].reshape(1, _CELLS, H)
    ap = stc_ref[2 * _CELLS:3 * _CELLS].reshape(1, _CELLS, H)
    am_ = stc_ref[3 * _CELLS:4 * _CELLS].reshape(1, _CELLS, H)
    rdt = stc_ref[4 * _CELLS:5 * _CELLS].reshape(1, _CELLS, H)

    z1 = jnp.zeros((1, H), f32)
    z12 = jnp.zeros((_S, H), f32)

    W_in = W_in_ref[...]
    b_in = b_in_ref[...]
    Wg1 = Wg1_ref[...]
    bg1 = bg1_ref[...]

    h1 = _elu(jnp.dot(map_ref[...], W_in, preferred_element_type=f32) + b_in)
    h1_meta = _elu(b_in)

    up = jnp.concatenate([h1[1:], z1], axis=0).reshape(BB, _CELLS, H)
    dn = jnp.concatenate([z1, h1[:-1]], axis=0).reshape(BB, _CELLS, H)
    xp = jnp.concatenate([h1[_S:], z12], axis=0).reshape(BB, _CELLS, H)
    xm = jnp.concatenate([z12, h1[:-_S]], axis=0).reshape(BB, _CELLS, H)
    m_term = h1_meta.reshape(1, 1, H) * rdt
    rows1 = (h1.reshape(BB, _CELLS, H)
             + up * au + dn * ad + xp * ap + xm * am_ + m_term).reshape(N, H)
    sum1 = h1.reshape(BB, _CELLS, H).sum(axis=1)
    h2 = _elu(jnp.dot(rows1, Wg1, preferred_element_type=f32) + bg1)
    h2_meta = _elu(jnp.dot(h1_meta + sum1 * (1.0 / _CELLS), Wg1,
                           preferred_element_type=f32) + bg1)
    h2_ref[...] = h2

    sum2 = h2.reshape(BB, _CELLS, H).sum(axis=1)
    meta_r2 = h2_meta + sum2 * (1.0 / _CELLS)

    rj_rows = []
    for g in range(BB):
        base = g * _CELLS
        jg = j_ref[g, 0]
        yg = lax.rem(jg, _S)
        xg = lax.div(jg, _S)
        ctr = h2_ref[pl.ds(base + jg, 1), :]
        gu = h2_ref[pl.ds(base + jnp.minimum(jg + 1, _CELLS - 1), 1), :]
        gd = h2_ref[pl.ds(base + jnp.maximum(jg - 1, 0), 1), :]
        gr = h2_ref[pl.ds(base + jnp.minimum(jg + _S, _CELLS - 1), 1), :]
        gl = h2_ref[pl.ds(base + jnp.maximum(jg - _S, 0), 1), :]
        fu = (yg < _S - 1).astype(f32)
        fd = (yg > 0).astype(f32)
        fr = (xg < _S - 1).astype(f32)
        fl = (xg > 0).astype(f32)
        nb = gu * fu + gd * fd + gr * fr + gl * fl + h2_meta[g:g + 1, :]
        rj_rows.append(ctr + nb * (1.0 / (1.0 + fu + fd + fr + fl)))
    rj = jnp.concatenate(rj_rows, axis=0)

    st = jnp.concatenate([rj, meta_r2], axis=0)
    h3 = _elu(jnp.dot(st, Wg2_ref[...], preferred_element_type=f32)
              + bg2_ref[...])
    emb = jnp.dot(h3, W_out_ref[...], preferred_element_type=f32) \
        + b_out_ref[...]
    cs = emb[:BB]
    ms = emb[BB:]

    pt = piece_ref[...]
    p0 = jnp.floor(pt[:, 0:1])
    oh3 = jnp.concatenate([(p0 == 0.0).astype(f32), (p0 == 1.0).astype(f32),
                           (p0 == 2.0).astype(f32)], axis=1)
    P1 = P1_ref[...]
    hp = _elu(jnp.dot(cs, P1[:128], preferred_element_type=f32)
              + jnp.dot(oh3, P1[128:131], preferred_element_type=f32)
              + p1b_ref[...])
    hp = _elu(jnp.dot(hp, P2_ref[...], preferred_element_type=f32)
              + p2b_ref[...])
    logits = jnp.dot(hp, P3_ref[...], preferred_element_type=f32) \
        + p3b_ref[...]
    lm = jnp.where(pt[:, 3:16] != 0.0, logits, _NEG)
    logits_ref[...] = lm

    am = lm + gum_ref[...]
    mx = jnp.max(am, axis=1, keepdims=True)
    li = lax.broadcasted_iota(jnp.int32, am.shape, 1)
    act_ref[...] = jnp.min(jnp.where(am == mx, li, am.shape[1]),
                           axis=1, keepdims=True).astype(jnp.int32)

    hv = _elu(jnp.dot(ms, V1_ref[...], preferred_element_type=f32)
              + v1b_ref[...])
    hv = _elu(jnp.dot(hv, V2_ref[...], preferred_element_type=f32)
              + v2b_ref[...])
    val_ref[...] = jnp.tanh(
        jnp.dot(hv, V3_ref[...], preferred_element_type=f32) + v3b_ref[...])


def kernel(map_tensor, piece_tensor, W_in, b_in, Wg1, bg1, Wg2, bg2, W_out,
           b_out, P1, p1b, P2, p2b, P3, p3b, V1, v1b, V2, v2b, V3, v3b):
    B = map_tensor.shape[0]
    assert B % _BB == 0
    n_act = P3.shape[1]
    map_flat = map_tensor.reshape(B * _CELLS, map_tensor.shape[-1])
    gum = jax.random.gumbel(jax.random.key(1), (B, n_act), jnp.float32)
    pos = piece_tensor[:, 1:3].astype(jnp.int32)
    jarr = (pos[:, 0] * _S + pos[:, 1]).reshape(B, 1)

    row2 = lambda v: v.reshape(1, -1)
    weights = (W_in, row2(b_in), Wg1, row2(bg1), Wg2, row2(bg2),
               W_out, row2(b_out), P1, row2(p1b), P2, row2(p2b),
               P3, row2(p3b), V1, row2(v1b), V2, row2(v2b),
               V3, row2(v3b))

    w_specs = [pl.BlockSpec(w.shape, lambda i: (0, 0)) for w in weights]
    grid = (B // _BB,)
    logits_m, act2d, value = pl.pallas_call(
        _fused_kernel,
        grid=grid,
        in_specs=[
            pl.BlockSpec((_BB * _CELLS, map_flat.shape[1]), lambda i: (i, 0)),
            pl.BlockSpec((_BB, piece_tensor.shape[1]), lambda i: (i, 0)),
            pl.BlockSpec((_BB, n_act), lambda i: (i, 0)),
            pl.BlockSpec((_BB, 1), lambda i: (i, 0),
                         memory_space=pltpu.SMEM),
            pl.BlockSpec(_STC.shape, lambda i: (0, 0)),
        ] + w_specs,
        out_specs=[
            pl.BlockSpec((_BB, n_act), lambda i: (i, 0)),
            pl.BlockSpec((_BB, 1), lambda i: (i, 0)),
            pl.BlockSpec((_BB, 1), lambda i: (i, 0)),
        ],
        out_shape=[
            jax.ShapeDtypeStruct((B, n_act), jnp.float32),
            jax.ShapeDtypeStruct((B, 1), jnp.int32),
            jax.ShapeDtypeStruct((B, 1), jnp.float32),
        ],
        scratch_shapes=[pltpu.VMEM((_BB * _CELLS, 128), jnp.float32)],
    )(map_flat, piece_tensor, gum, jarr, jnp.asarray(_STC), *weights)
    return (act2d.reshape(B), logits_m, value)


# speedup vs baseline: 31.7243x; 1.0003x over previous
import jax
import jax.numpy as jnp
import numpy as np
from jax import lax
from jax.experimental import pallas as pl
from jax.experimental.pallas import tpu as pltpu

_S = 12
_CELLS = _S * _S
_BB = 64
_NEG = jnp.finfo(jnp.float32).min


def _stencil_consts():
    c = np.arange(_CELLS)
    y, x = c % _S, c // _S
    my1 = (y < _S - 1).astype(np.float32)
    my0 = (y > 0).astype(np.float32)
    mx1 = (x < _S - 1).astype(np.float32)
    mx0 = (x > 0).astype(np.float32)
    rdeg = 1.0 / (1.0 + my1 + my0 + mx1 + mx0)
    rows = np.concatenate([my1 * rdeg, my0 * rdeg, mx1 * rdeg, mx0 * rdeg,
                           rdeg]).astype(np.float32)
    return np.repeat(rows[:, None], 128, axis=1)


_STC = _stencil_consts()


def _elu(v):
    return jnp.maximum(v, 0.0) + jnp.exp(jnp.minimum(v, 0.0)) - 1.0


def _fused_kernel(map_ref, piece_ref, gum_ref, j_ref, stc_ref,
                  W_in_ref, b_in_ref, Wg1_ref, bg1_ref, Wg2_ref, bg2_ref,
                  W_out_ref, b_out_ref, P1_ref, p1b_ref, P2_ref, p2b_ref,
                  P3_ref, p3b_ref, V1_ref, v1b_ref, V2_ref, v2b_ref,
                  V3_ref, v3b_ref,
                  logits_ref, act_ref, val_ref,
                  h2_ref):
    BB = _BB
    N = BB * _CELLS
    H = 128
    f32 = jnp.float32

    au = stc_ref[0:_CELLS].reshape(1, _CELLS, H)
    ad = stc_ref[_CELLS:2 * _CELLS].reshape(1, _CELLS, H)
    ap = stc_ref[2 * _CELLS:3 * _CELLS].reshape(1, _CELLS, H)
    am_ = stc_ref[3 * _CELLS:4 * _CELLS].reshape(1, _CELLS, H)
    rdt = stc_ref[4 * _CELLS:5 * _CELLS].reshape(1, _CELLS, H)

    z1 = jnp.zeros((1, H), f32)
    z12 = jnp.zeros((_S, H), f32)

    W_in = W_in_ref[...]
    b_in = b_in_ref[...]
    Wg1 = Wg1_ref[...]
    bg1 = bg1_ref[...]

    h1 = _elu(jnp.dot(map_ref[...], W_in, preferred_element_type=f32) + b_in)
    h1_meta = _elu(b_in)

    up = jnp.concatenate([h1[1:], z1], axis=0).reshape(BB, _CELLS, H)
    dn = jnp.concatenate([z1, h1[:-1]], axis=0).reshape(BB, _CELLS, H)
    xp = jnp.concatenate([h1[_S:], z12], axis=0).reshape(BB, _CELLS, H)
    xm = jnp.concatenate([z12, h1[:-_S]], axis=0).reshape(BB, _CELLS, H)
    m_term = h1_meta.reshape(1, 1, H) * rdt
    rows1 = (h1.reshape(BB, _CELLS, H)
             + up * au + dn * ad + xp * ap + xm * am_ + m_term).reshape(N, H)
    sum1 = h1.reshape(BB, _CELLS, H).sum(axis=1)
    h2 = _elu(jnp.dot(rows1, Wg1, preferred_element_type=f32) + bg1)
    h2_meta = _elu(jnp.dot(h1_meta + sum1 * (1.0 / _CELLS), Wg1,
                           preferred_element_type=f32) + bg1)
    h2_ref[...] = h2

    sum2 = h2.reshape(BB, _CELLS, H).sum(axis=1)
    meta_r2 = h2_meta + sum2 * (1.0 / _CELLS)

    rj_rows = []
    for g in range(BB):
        base = g * _CELLS
        jg = j_ref[g, 0]
        yg = lax.rem(jg, _S)
        xg = lax.div(jg, _S)
        ctr = h2_ref[pl.ds(base + jg, 1), :]
        gu = h2_ref[pl.ds(base + jnp.minimum(jg + 1, _CELLS - 1), 1), :]
        gd = h2_ref[pl.ds(base + jnp.maximum(jg - 1, 0), 1), :]
        gr = h2_ref[pl.ds(base + jnp.minimum(jg + _S, _CELLS - 1), 1), :]
        gl = h2_ref[pl.ds(base + jnp.maximum(jg - _S, 0), 1), :]
        fu = (yg < _S - 1).astype(f32)
        fd = (yg > 0).astype(f32)
        fr = (xg < _S - 1).astype(f32)
        fl = (xg > 0).astype(f32)
        nb = gu * fu + gd * fd + gr * fr + gl * fl + h2_meta[g:g + 1, :]
        rj_rows.append(ctr + nb * (1.0 / (1.0 + fu + fd + fr + fl)))
    rj = jnp.concatenate(rj_rows, axis=0)

    st = jnp.concatenate([rj, meta_r2], axis=0)
    h3 = _elu(jnp.dot(st, Wg2_ref[...], preferred_element_type=f32)
              + bg2_ref[...])
    emb = jnp.dot(h3, W_out_ref[...], preferred_element_type=f32) \
        + b_out_ref[...]
    cs = emb[:BB]
    ms = emb[BB:]

    pt = piece_ref[...]
    p0 = jnp.floor(pt[:, 0:1])
    oh3 = jnp.concatenate([(p0 == 0.0).astype(f32), (p0 == 1.0).astype(f32),
                           (p0 == 2.0).astype(f32)], axis=1)
    P1 = P1_ref[...]
    hp = _elu(jnp.dot(cs, P1[:128], preferred_element_type=f32)
              + jnp.dot(oh3, P1[128:131], preferred_element_type=f32)
              + p1b_ref[...])
    hp = _elu(jnp.dot(hp, P2_ref[...], preferred_element_type=f32)
              + p2b_ref[...])
    logits = jnp.dot(hp, P3_ref[...], preferred_element_type=f32) \
        + p3b_ref[...]
    lm = jnp.where(pt[:, 3:16] != 0.0, logits, _NEG)
    logits_ref[...] = lm

    am = lm + gum_ref[...]
    mx = jnp.max(am, axis=1, keepdims=True)
    li = lax.broadcasted_iota(jnp.int32, am.shape, 1)
    act_ref[...] = jnp.min(jnp.where(am == mx, li, am.shape[1]),
                           axis=1, keepdims=True).astype(jnp.int32)

    hv = _elu(jnp.dot(ms, V1_ref[...], preferred_element_type=f32)
              + v1b_ref[...])
    hv = _elu(jnp.dot(hv, V2_ref[...], preferred_element_type=f32)
              + v2b_ref[...])
    val_ref[...] = jnp.tanh(
        jnp.dot(hv, V3_ref[...], preferred_element_type=f32) + v3b_ref[...])


def kernel(map_tensor, piece_tensor, W_in, b_in, Wg1, bg1, Wg2, bg2, W_out,
           b_out, P1, p1b, P2, p2b, P3, p3b, V1, v1b, V2, v2b, V3, v3b):
    B = map_tensor.shape[0]
    assert B % _BB == 0
    n_act = P3.shape[1]
    map_flat = map_tensor.reshape(B * _CELLS, map_tensor.shape[-1])
    gum = jax.random.gumbel(jax.random.key(1), (B, n_act), jnp.float32)
    pos = piece_tensor[:, 1:3].astype(jnp.int32)
    jarr = (pos[:, 0] * _S + pos[:, 1]).reshape(B, 1)

    row2 = lambda v: v.reshape(1, -1)
    weights = (W_in, row2(b_in), Wg1, row2(bg1), Wg2, row2(bg2),
               W_out, row2(b_out), P1, row2(p1b), P2, row2(p2b),
               P3, row2(p3b), V1, row2(v1b), V2, row2(v2b),
               V3, row2(v3b))

    w_specs = [pl.BlockSpec(w.shape, lambda i: (0, 0)) for w in weights]
    grid = (B // _BB,)
    logits_m, act2d, value = pl.pallas_call(
        _fused_kernel,
        grid=grid,
        in_specs=[
            pl.BlockSpec((_BB * _CELLS, map_flat.shape[1]), lambda i: (i, 0)),
            pl.BlockSpec((_BB, piece_tensor.shape[1]), lambda i: (i, 0)),
            pl.BlockSpec((_BB, n_act), lambda i: (i, 0)),
            pl.BlockSpec((_BB, 1), lambda i: (i, 0),
                         memory_space=pltpu.SMEM),
            pl.BlockSpec(_STC.shape, lambda i: (0, 0)),
        ] + w_specs,
        out_specs=[
            pl.BlockSpec((_BB, n_act), lambda i: (i, 0)),
            pl.BlockSpec((_BB, 1), lambda i: (i, 0)),
            pl.BlockSpec((_BB, 1), lambda i: (i, 0)),
        ],
        out_shape=[
            jax.ShapeDtypeStruct((B, n_act), jnp.float32),
            jax.ShapeDtypeStruct((B, 1), jnp.int32),
            jax.ShapeDtypeStruct((B, 1), jnp.float32),
        ],
        scratch_shapes=[pltpu.VMEM((_BB * _CELLS, 128), jnp.float32)],
        compiler_params=pltpu.CompilerParams(
            dimension_semantics=("parallel",)),
    )(map_flat, piece_tensor, gum, jarr, jnp.asarray(_STC), *weights)
    return (act2d.reshape(B), logits_m, value)


# speedup vs baseline: 34.3160x; 1.0817x over previous
import jax
import jax.numpy as jnp
import numpy as np
from jax import lax
from jax.experimental import pallas as pl
from jax.experimental.pallas import tpu as pltpu

_S = 12
_CELLS = _S * _S
_BB = 64
_NEG = jnp.finfo(jnp.float32).min


def _stencil_consts():
    c = np.arange(_CELLS)
    y, x = c % _S, c // _S
    my1 = (y < _S - 1).astype(np.float32)
    my0 = (y > 0).astype(np.float32)
    mx1 = (x < _S - 1).astype(np.float32)
    mx0 = (x > 0).astype(np.float32)
    rdeg = 1.0 / (1.0 + my1 + my0 + mx1 + mx0)
    rows = np.concatenate([my1 * rdeg, my0 * rdeg, mx1 * rdeg, mx0 * rdeg,
                           rdeg]).astype(np.float32)
    return np.repeat(rows[:, None], 128, axis=1)


_STC = _stencil_consts()


def _elu(v):
    return jnp.maximum(v, 0.0) + jnp.exp(jnp.minimum(v, 0.0)) - 1.0


def _fused_kernel(map_ref, piece_ref, gum_ref, j_ref, stc_ref,
                  W_in_ref, b_in_ref, Wg1_ref, bg1_ref, Wg2_ref, bg2_ref,
                  W_out_ref, b_out_ref, P1_ref, p1b_ref, P2_ref, p2b_ref,
                  P3_ref, p3b_ref, V1_ref, v1b_ref, V2_ref, v2b_ref,
                  V3_ref, v3b_ref,
                  logits_ref, act_ref, val_ref,
                  h2_ref, h1p_ref):
    BB = _BB
    N = BB * _CELLS
    H = 128
    f32 = jnp.float32

    au = stc_ref[0:_CELLS].reshape(1, _CELLS, H)
    ad = stc_ref[_CELLS:2 * _CELLS].reshape(1, _CELLS, H)
    ap = stc_ref[2 * _CELLS:3 * _CELLS].reshape(1, _CELLS, H)
    am_ = stc_ref[3 * _CELLS:4 * _CELLS].reshape(1, _CELLS, H)
    rdt = stc_ref[4 * _CELLS:5 * _CELLS].reshape(1, _CELLS, H)

    W_in = W_in_ref[...]
    b_in = b_in_ref[...]
    Wg1 = Wg1_ref[...]
    bg1 = bg1_ref[...]

    h1 = _elu(jnp.dot(map_ref[...], W_in, preferred_element_type=f32) + b_in)
    h1_meta = _elu(b_in)

    h1p_ref[pl.ds(0, _S)] = jnp.zeros((_S, H), f32)
    h1p_ref[pl.ds(_S, N)] = h1
    h1p_ref[pl.ds(_S + N, _S)] = jnp.zeros((_S, H), f32)
    up = h1p_ref[pl.ds(_S + 1, N)].reshape(BB, _CELLS, H)
    dn = h1p_ref[pl.ds(_S - 1, N)].reshape(BB, _CELLS, H)
    xp = h1p_ref[pl.ds(2 * _S, N)].reshape(BB, _CELLS, H)
    xm = h1p_ref[pl.ds(0, N)].reshape(BB, _CELLS, H)
    rows1 = (h1.reshape(BB, _CELLS, H)
             + up * au + dn * ad + xp * ap + xm * am_).reshape(N, H)
    sum1 = h1.reshape(BB, _CELLS, H).sum(axis=1)
    bias2 = (jnp.dot(h1_meta.reshape(1, H) * rdt.reshape(_CELLS, H),
                     Wg1, preferred_element_type=f32) + bg1)
    h2 = _elu((jnp.dot(rows1, Wg1, preferred_element_type=f32)
               .reshape(BB, _CELLS, H)
               + bias2.reshape(1, _CELLS, H)).reshape(N, H))
    h2_meta = _elu(jnp.dot(h1_meta + sum1 * (1.0 / _CELLS), Wg1,
                           preferred_element_type=f32) + bg1)
    h2_ref[...] = h2

    sum2 = h2.reshape(BB, _CELLS, H).sum(axis=1)
    meta_r2 = h2_meta + sum2 * (1.0 / _CELLS)

    rj_rows = []
    for g in range(BB):
        base = g * _CELLS
        jg = j_ref[g, 0]
        yg = lax.rem(jg, _S)
        xg = lax.div(jg, _S)
        ctr = h2_ref[pl.ds(base + jg, 1), :]
        gu = h2_ref[pl.ds(base + jnp.minimum(jg + 1, _CELLS - 1), 1), :]
        gd = h2_ref[pl.ds(base + jnp.maximum(jg - 1, 0), 1), :]
        gr = h2_ref[pl.ds(base + jnp.minimum(jg + _S, _CELLS - 1), 1), :]
        gl = h2_ref[pl.ds(base + jnp.maximum(jg - _S, 0), 1), :]
        fu = (yg < _S - 1).astype(f32)
        fd = (yg > 0).astype(f32)
        fr = (xg < _S - 1).astype(f32)
        fl = (xg > 0).astype(f32)
        nb = gu * fu + gd * fd + gr * fr + gl * fl + h2_meta[g:g + 1, :]
        rj_rows.append(ctr + nb * (1.0 / (1.0 + fu + fd + fr + fl)))
    rj = jnp.concatenate(rj_rows, axis=0)

    st = jnp.concatenate([rj, meta_r2], axis=0)
    h3 = _elu(jnp.dot(st, Wg2_ref[...], preferred_element_type=f32)
              + bg2_ref[...])
    emb = jnp.dot(h3, W_out_ref[...], preferred_element_type=f32) \
        + b_out_ref[...]
    cs = emb[:BB]
    ms = emb[BB:]

    pt = piece_ref[...]
    p0 = jnp.floor(pt[:, 0:1])
    oh3 = jnp.concatenate([(p0 == 0.0).astype(f32), (p0 == 1.0).astype(f32),
                           (p0 == 2.0).astype(f32)], axis=1)
    P1 = P1_ref[...]
    hp = _elu(jnp.dot(cs, P1[:128], preferred_element_type=f32)
              + jnp.dot(oh3, P1[128:131], preferred_element_type=f32)
              + p1b_ref[...])
    hp = _elu(jnp.dot(hp, P2_ref[...], preferred_element_type=f32)
              + p2b_ref[...])
    logits = jnp.dot(hp, P3_ref[...], preferred_element_type=f32) \
        + p3b_ref[...]
    lm = jnp.where(pt[:, 3:16] != 0.0, logits, _NEG)
    logits_ref[...] = lm

    am = lm + gum_ref[...]
    mx = jnp.max(am, axis=1, keepdims=True)
    li = lax.broadcasted_iota(jnp.int32, am.shape, 1)
    act_ref[...] = jnp.min(jnp.where(am == mx, li, am.shape[1]),
                           axis=1, keepdims=True).astype(jnp.int32)

    hv = _elu(jnp.dot(ms, V1_ref[...], preferred_element_type=f32)
              + v1b_ref[...])
    hv = _elu(jnp.dot(hv, V2_ref[...], preferred_element_type=f32)
              + v2b_ref[...])
    val_ref[...] = jnp.tanh(
        jnp.dot(hv, V3_ref[...], preferred_element_type=f32) + v3b_ref[...])


def kernel(map_tensor, piece_tensor, W_in, b_in, Wg1, bg1, Wg2, bg2, W_out,
           b_out, P1, p1b, P2, p2b, P3, p3b, V1, v1b, V2, v2b, V3, v3b):
    B = map_tensor.shape[0]
    assert B % _BB == 0
    n_act = P3.shape[1]
    map_flat = map_tensor.reshape(B * _CELLS, map_tensor.shape[-1])
    gum = jax.random.gumbel(jax.random.key(1), (B, n_act), jnp.float32)
    pos = piece_tensor[:, 1:3].astype(jnp.int32)
    jarr = (pos[:, 0] * _S + pos[:, 1]).reshape(B, 1)

    row2 = lambda v: v.reshape(1, -1)
    weights = (W_in, row2(b_in), Wg1, row2(bg1), Wg2, row2(bg2),
               W_out, row2(b_out), P1, row2(p1b), P2, row2(p2b),
               P3, row2(p3b), V1, row2(v1b), V2, row2(v2b),
               V3, row2(v3b))

    w_specs = [pl.BlockSpec(w.shape, lambda i: (0, 0)) for w in weights]
    grid = (B // _BB,)
    logits_m, act2d, value = pl.pallas_call(
        _fused_kernel,
        grid=grid,
        in_specs=[
            pl.BlockSpec((_BB * _CELLS, map_flat.shape[1]), lambda i: (i, 0)),
            pl.BlockSpec((_BB, piece_tensor.shape[1]), lambda i: (i, 0)),
            pl.BlockSpec((_BB, n_act), lambda i: (i, 0)),
            pl.BlockSpec((_BB, 1), lambda i: (i, 0),
                         memory_space=pltpu.SMEM),
            pl.BlockSpec(_STC.shape, lambda i: (0, 0)),
        ] + w_specs,
        out_specs=[
            pl.BlockSpec((_BB, n_act), lambda i: (i, 0)),
            pl.BlockSpec((_BB, 1), lambda i: (i, 0)),
            pl.BlockSpec((_BB, 1), lambda i: (i, 0)),
        ],
        out_shape=[
            jax.ShapeDtypeStruct((B, n_act), jnp.float32),
            jax.ShapeDtypeStruct((B, 1), jnp.int32),
            jax.ShapeDtypeStruct((B, 1), jnp.float32),
        ],
        scratch_shapes=[pltpu.VMEM((_BB * _CELLS, 128), jnp.float32),
                        pltpu.VMEM((_BB * _CELLS + 2 * _S, 128), jnp.float32)],
        compiler_params=pltpu.CompilerParams(
            dimension_semantics=("parallel",)),
    )(map_flat, piece_tensor, gum, jarr, jnp.asarray(_STC), *weights)
    return (act2d.reshape(B), logits_m, value)


# speedup vs baseline: 35.1890x; 1.0254x over previous
import jax
import jax.numpy as jnp
import numpy as np
from jax import lax
from jax.experimental import pallas as pl
from jax.experimental.pallas import tpu as pltpu

_S = 12
_CELLS = _S * _S
_BB = 64
_NEG = jnp.finfo(jnp.float32).min


def _stencil_consts():
    c = np.arange(_CELLS)
    y, x = c % _S, c // _S
    my1 = (y < _S - 1).astype(np.float32)
    my0 = (y > 0).astype(np.float32)
    mx1 = (x < _S - 1).astype(np.float32)
    mx0 = (x > 0).astype(np.float32)
    rdeg = (1.0 / (1.0 + my1 + my0 + mx1 + mx0)).astype(np.float32)
    rows = np.concatenate([my1 * rdeg, my0 * rdeg, mx1 * rdeg,
                           mx0 * rdeg]).astype(np.float32)
    return np.repeat(rows[:, None], 128, axis=1), rdeg


_STC, _RDEG = _stencil_consts()


def _elu(v):
    return jnp.maximum(v, 0.0) + jnp.exp(jnp.minimum(v, 0.0)) - 1.0


def _elup1(v):
    return jnp.maximum(v, 0.0) + jnp.exp(jnp.minimum(v, 0.0))


def _fused_kernel(map_ref, piece_ref, gum_ref, j_ref, stc_ref,
                  W_in_ref, b_in_ref, Wg1_ref, bmeta_ref, bias2_ref,
                  Wg2_ref, bg2_ref,
                  W_out_ref, b_out_ref, P1_ref, p1b_ref, P2_ref, p2b_ref,
                  P3_ref, p3b_ref, V1_ref, v1b_ref, V2_ref, v2b_ref,
                  V3_ref, v3b_ref,
                  logits_ref, act_ref, val_ref,
                  h2_ref, h1p_ref):
    BB = _BB
    N = BB * _CELLS
    H = 128
    f32 = jnp.float32

    au = stc_ref[0:_CELLS].reshape(1, _CELLS, H)
    ad = stc_ref[_CELLS:2 * _CELLS].reshape(1, _CELLS, H)
    ap = stc_ref[2 * _CELLS:3 * _CELLS].reshape(1, _CELLS, H)
    am_ = stc_ref[3 * _CELLS:4 * _CELLS].reshape(1, _CELLS, H)

    W_in = W_in_ref[...]
    b_in = b_in_ref[...]
    Wg1 = Wg1_ref[...]

    h1 = _elup1(jnp.dot(map_ref[...], W_in, preferred_element_type=f32)
                + b_in)

    h1p_ref[pl.ds(0, _S)] = jnp.zeros((_S, H), f32)
    h1p_ref[pl.ds(_S, N)] = h1
    h1p_ref[pl.ds(_S + N, _S)] = jnp.zeros((_S, H), f32)
    up = h1p_ref[pl.ds(_S + 1, N)].reshape(BB, _CELLS, H)
    dn = h1p_ref[pl.ds(_S - 1, N)].reshape(BB, _CELLS, H)
    xp = h1p_ref[pl.ds(2 * _S, N)].reshape(BB, _CELLS, H)
    xm = h1p_ref[pl.ds(0, N)].reshape(BB, _CELLS, H)
    rows1 = (h1.reshape(BB, _CELLS, H)
             + up * au + dn * ad + xp * ap + xm * am_).reshape(N, H)
    sum1 = h1.reshape(BB, _CELLS, H).sum(axis=1)
    h2 = _elup1((jnp.dot(rows1, Wg1, preferred_element_type=f32)
                 .reshape(BB, _CELLS, H)
                 + bias2_ref[...].reshape(1, _CELLS, H)).reshape(N, H))
    h2_meta = _elup1(jnp.dot(sum1 * (1.0 / _CELLS), Wg1,
                             preferred_element_type=f32)
                     + bmeta_ref[...])
    h2_ref[...] = h2

    sum2 = h2.reshape(BB, _CELLS, H).sum(axis=1)
    meta_r2 = h2_meta + sum2 * (1.0 / _CELLS)

    rj_rows = []
    for g in range(BB):
        base = g * _CELLS
        jg = j_ref[g, 0]
        yg = lax.rem(jg, _S)
        xg = lax.div(jg, _S)
        ctr = h2_ref[pl.ds(base + jg, 1), :]
        gu = h2_ref[pl.ds(base + jnp.minimum(jg + 1, _CELLS - 1), 1), :]
        gd = h2_ref[pl.ds(base + jnp.maximum(jg - 1, 0), 1), :]
        gr = h2_ref[pl.ds(base + jnp.minimum(jg + _S, _CELLS - 1), 1), :]
        gl = h2_ref[pl.ds(base + jnp.maximum(jg - _S, 0), 1), :]
        fu = (yg < _S - 1).astype(f32)
        fd = (yg > 0).astype(f32)
        fr = (xg < _S - 1).astype(f32)
        fl = (xg > 0).astype(f32)
        nb = gu * fu + gd * fd + gr * fr + gl * fl + h2_meta[g:g + 1, :]
        rj_rows.append(ctr + nb * (1.0 / (1.0 + fu + fd + fr + fl)))
    rj = jnp.concatenate(rj_rows, axis=0)

    st = jnp.concatenate([rj, meta_r2], axis=0)
    h3 = _elup1(jnp.dot(st, Wg2_ref[...], preferred_element_type=f32)
                + bg2_ref[...])
    emb = jnp.dot(h3, W_out_ref[...], preferred_element_type=f32) \
        + b_out_ref[...]
    cs = emb[:BB]
    ms = emb[BB:]

    pt = piece_ref[...]
    p0 = jnp.floor(pt[:, 0:1])
    oh3 = jnp.concatenate([(p0 == 0.0).astype(f32), (p0 == 1.0).astype(f32),
                           (p0 == 2.0).astype(f32)], axis=1)
    P1 = P1_ref[...]
    hp = _elu(jnp.dot(cs, P1[:128], preferred_element_type=f32)
              + jnp.dot(oh3, P1[128:131], preferred_element_type=f32)
              + p1b_ref[...])
    hp = _elu(jnp.dot(hp, P2_ref[...], preferred_element_type=f32)
              + p2b_ref[...])
    logits = jnp.dot(hp, P3_ref[...], preferred_element_type=f32) \
        + p3b_ref[...]
    lm = jnp.where(pt[:, 3:16] != 0.0, logits, _NEG)
    logits_ref[...] = lm

    am = lm + gum_ref[...]
    mx = jnp.max(am, axis=1, keepdims=True)
    li = lax.broadcasted_iota(jnp.int32, am.shape, 1)
    act_ref[...] = jnp.min(jnp.where(am == mx, li, am.shape[1]),
                           axis=1, keepdims=True).astype(jnp.int32)

    hv = _elu(jnp.dot(ms, V1_ref[...], preferred_element_type=f32)
              + v1b_ref[...])
    hv = _elu(jnp.dot(hv, V2_ref[...], preferred_element_type=f32)
              + v2b_ref[...])
    val_ref[...] = jnp.tanh(
        jnp.dot(hv, V3_ref[...], preferred_element_type=f32) + v3b_ref[...])


def kernel(map_tensor, piece_tensor, W_in, b_in, Wg1, bg1, Wg2, bg2, W_out,
           b_out, P1, p1b, P2, p2b, P3, p3b, V1, v1b, V2, v2b, V3, v3b):
    B = map_tensor.shape[0]
    assert B % _BB == 0
    n_act = P3.shape[1]
    map_flat = map_tensor.reshape(B * _CELLS, map_tensor.shape[-1])
    gum = jax.random.gumbel(jax.random.key(1), (B, n_act), jnp.float32)
    pos = piece_tensor[:, 1:3].astype(jnp.int32)
    jarr = (pos[:, 0] * _S + pos[:, 1]).reshape(B, 1)

    row2 = lambda v: v.reshape(1, -1)
    hm = jax.nn.elu(b_in)
    colW1 = Wg1.sum(axis=0)
    rdeg = jnp.asarray(_RDEG)
    bias2eff = (bg1[None, :] + rdeg[:, None] * (hm @ Wg1)[None, :]
                - (2.0 - rdeg)[:, None] * colW1[None, :])
    bmeta = row2((hm - 1.0) @ Wg1 + bg1)
    bg2eff = row2(bg2 - 2.0 * Wg2.sum(axis=0))
    bouteff = row2(b_out - W_out.sum(axis=0))
    weights = (W_in, row2(b_in), Wg1, bmeta, bias2eff, Wg2, bg2eff,
               W_out, bouteff, P1, row2(p1b), P2, row2(p2b),
               P3, row2(p3b), V1, row2(v1b), V2, row2(v2b),
               V3, row2(v3b))

    w_specs = [pl.BlockSpec(w.shape, lambda i: (0, 0)) for w in weights]
    grid = (B // _BB,)
    logits_m, act2d, value = pl.pallas_call(
        _fused_kernel,
        grid=grid,
        in_specs=[
            pl.BlockSpec((_BB * _CELLS, map_flat.shape[1]), lambda i: (i, 0)),
            pl.BlockSpec((_BB, piece_tensor.shape[1]), lambda i: (i, 0)),
            pl.BlockSpec((_BB, n_act), lambda i: (i, 0)),
            pl.BlockSpec((_BB, 1), lambda i: (i, 0),
                         memory_space=pltpu.SMEM),
            pl.BlockSpec(_STC.shape, lambda i: (0, 0)),
        ] + w_specs,
        out_specs=[
            pl.BlockSpec((_BB, n_act), lambda i: (i, 0)),
            pl.BlockSpec((_BB, 1), lambda i: (i, 0)),
            pl.BlockSpec((_BB, 1), lambda i: (i, 0)),
        ],
        out_shape=[
            jax.ShapeDtypeStruct((B, n_act), jnp.float32),
            jax.ShapeDtypeStruct((B, 1), jnp.int32),
            jax.ShapeDtypeStruct((B, 1), jnp.float32),
        ],
        scratch_shapes=[pltpu.VMEM((_BB * _CELLS, 128), jnp.float32),
                        pltpu.VMEM((_BB * _CELLS + 2 * _S, 128), jnp.float32)],
        compiler_params=pltpu.CompilerParams(
            dimension_semantics=("parallel",)),
    )(map_flat, piece_tensor, gum, jarr, jnp.asarray(_STC), *weights)
    return (act2d.reshape(B), logits_m, value)
